# Initial kernel scaffold; baseline (speedup 1.0000x reference)
import jax
import jax.numpy as jnp
from jax.experimental import pallas as pl


def kernel(x, internal_edge_index, internal_edge_attr, edge_index, edge_attr, batch, params):
    raise NotImplementedError("write your pallas kernel here")


# speedup vs baseline: 178.6595x; 178.6595x over previous
import functools

import jax
import jax.numpy as jnp
from jax import lax
from jax.experimental import pallas as pl
from jax.experimental.pallas import tpu as pltpu
from jax.experimental.pallas import tpu_sc as plsc

_N = 10000
_E = 320000
_G = 16

_NC = 2
_NS = 16
_L = 16

_EPT = _E // _NS


def _hist_body(src_hbm, out_hbm, idx_v, ones_v, zeros_v, hist_sh):
    c = lax.axis_index("c")
    s = lax.axis_index("s")

    def fill(i, _):
        ones_v[pl.ds(i * _L, _L)] = jnp.full((_L,), 1.0, jnp.float32)
        return 0

    lax.fori_loop(0, _EPT // _L, fill, 0)

    def fill0(i, _):
        zeros_v[pl.ds(i * _L, _L)] = jnp.zeros((_L,), jnp.float32)
        return 0

    lax.fori_loop(0, _N // _L, fill0, 0)

    base = c * _E + s * _EPT
    pltpu.sync_copy(src_hbm.at[pl.ds(base, _EPT)], idx_v)

    @pl.when(s == 0)
    def _zero_hist():
        pltpu.sync_copy(zeros_v, hist_sh)

    plsc.subcore_barrier()

    pltpu.sync_copy(ones_v, hist_sh.at[idx_v], add=True)

    plsc.subcore_barrier()

    @pl.when(s == 0)
    def _writeback():
        pltpu.sync_copy(hist_sh, out_hbm.at[c])


@functools.cache
def _hist_kernel():
    return pl.kernel(
        _hist_body,
        out_type=jax.ShapeDtypeStruct((_NC, _N), jnp.float32),
        mesh=plsc.VectorSubcoreMesh(
            core_axis_name="c", subcore_axis_name="s",
            num_cores=_NC, num_subcores=_NS),
        scratch_types=[
            pltpu.VMEM((_EPT,), jnp.int32),
            pltpu.VMEM((_EPT,), jnp.float32),
            pltpu.VMEM((_N,), jnp.float32),
            pltpu.VMEM_SHARED((_N,), jnp.float32),
        ],
    )


def _dense_body(x_ref, deg_ref, batch_ref,
                w1i, b1i, w2i, b2i, w3i, b3i,
                w1e, b1e, w2e, b2e, w3e, b3e,
                fc1w, fc1b, fc2w, fc2b, out_ref):
    x = x_ref[...]
    relu = jax.nn.relu
    f32 = jnp.float32

    def branch(d, w1, b1, w2, b2, w3, b3):
        h = relu(d * (jnp.dot(x, w1[...], preferred_element_type=f32) + b1[...]))
        h = relu(d * (jnp.dot(h, w2[...], preferred_element_type=f32) + b2[...]))
        h = relu(d * (jnp.dot(h, w3[...], preferred_element_type=f32) + b3[...]))
        return h

    i3 = branch(deg_ref[:, 0:1], w1i, b1i, w2i, b2i, w3i, b3i)
    e3 = branch(deg_ref[:, 1:2], w1e, b1e, w2e, b2e, w3e, b3e)

    g_iota = lax.broadcasted_iota(jnp.int32, (_G, _N), 0)
    m = (batch_ref[...] == g_iota).astype(f32)
    cnt = jnp.maximum(jnp.sum(m, axis=1, keepdims=True), 1.0)
    ipg = jnp.dot(m, i3, preferred_element_type=f32) / cnt
    epg = jnp.dot(m, e3, preferred_element_type=f32) / cnt
    u = jnp.concatenate([ipg, epg], axis=1)
    o1 = relu(jnp.dot(u, fc1w[...], preferred_element_type=f32) + fc1b[...])
    out_ref[...] = jnp.dot(o1, fc2w[...], preferred_element_type=f32) + fc2b[...]


def kernel(x, internal_edge_index, internal_edge_attr, edge_index, edge_attr,
           batch, params):
    del internal_edge_attr, edge_attr

    srcs = jnp.concatenate(
        [internal_edge_index[0], edge_index[0]]).astype(jnp.int32)
    hist = _hist_kernel()(srcs)
    deg = hist.T

    p = params
    args = (
        x, deg, batch.astype(jnp.int32).reshape(1, _N),
        p["iml1"]["fn_w"].T, p["iml1"]["fn_b"].reshape(1, -1),
        p["iml2"]["fn_w"].T, p["iml2"]["fn_b"].reshape(1, -1),
        p["iml3"]["fn_w"].T, p["iml3"]["fn_b"].reshape(1, -1),
        p["eml1"]["fn_w"].T, p["eml1"]["fn_b"].reshape(1, -1),
        p["eml2"]["fn_w"].T, p["eml2"]["fn_b"].reshape(1, -1),
        p["eml3"]["fn_w"].T, p["eml3"]["fn_b"].reshape(1, -1),
        p["fc1_w"].T, p["fc1_b"].reshape(1, -1),
        p["fc2_w"].T, p["fc2_b"].reshape(1, -1),
    )
    return pl.pallas_call(
        _dense_body,
        out_shape=jax.ShapeDtypeStruct((_G, 8), jnp.float32),
    )(*args)


# speedup vs baseline: 241.5970x; 1.3523x over previous
import functools

import jax
import jax.numpy as jnp
from jax import lax
from jax.experimental import pallas as pl
from jax.experimental.pallas import tpu as pltpu
from jax.experimental.pallas import tpu_sc as plsc

_N = 10000
_E = 320000
_G = 16

_NC = 2
_NS = 16
_L = 16

_EPT = _E // _NS


def _hist_body(iei_hbm, eei_hbm, ones_hbm, zeros_hbm, out_hbm,
               idx_v, ones_v, hist_sh):
    c = lax.axis_index("c")
    s = lax.axis_index("s")

    pltpu.sync_copy(ones_hbm, ones_v)

    @pl.when(c == 0)
    def _load_internal():
        pltpu.sync_copy(iei_hbm.at[pl.ds(s * _EPT, _EPT)], idx_v)

    @pl.when(c == 1)
    def _load_external():
        pltpu.sync_copy(eei_hbm.at[pl.ds(s * _EPT, _EPT)], idx_v)

    @pl.when(s == 0)
    def _zero_hist():
        pltpu.sync_copy(zeros_hbm, hist_sh)

    plsc.subcore_barrier()

    pltpu.sync_copy(ones_v, hist_sh.at[idx_v], add=True)

    plsc.subcore_barrier()

    @pl.when(s == 0)
    def _writeback():
        pltpu.sync_copy(hist_sh, out_hbm.at[c])


@functools.cache
def _hist_kernel():
    return pl.kernel(
        _hist_body,
        out_type=jax.ShapeDtypeStruct((_NC, _N), jnp.float32),
        mesh=plsc.VectorSubcoreMesh(
            core_axis_name="c", subcore_axis_name="s",
            num_cores=_NC, num_subcores=_NS),
        scratch_types=[
            pltpu.VMEM((_EPT,), jnp.int32),
            pltpu.VMEM((_EPT,), jnp.float32),
            pltpu.VMEM_SHARED((_N,), jnp.float32),
        ],
    )


def _dense_body(x_ref, hist_ref, batch_ref,
                w1i, b1i, w2i, b2i, w3i, b3i,
                w1e, b1e, w2e, b2e, w3e, b3e,
                fc1w, fc1b, fc2w, fc2b, out_ref):
    x = x_ref[...]
    relu = jax.nn.relu
    f32 = jnp.float32

    def mm(a, b, dims):
        return lax.dot_general(a, b, (dims, ((), ())),
                               preferred_element_type=f32)

    def branch(d_row, w1, b1, w2, b2, w3, b3):
        h = relu(d_row * (mm(w1[...], x, ((1,), (1,))) + b1[...]))
        h = relu(d_row * (mm(w2[...], h, ((1,), (0,))) + b2[...]))
        h = relu(d_row * (mm(w3[...], h, ((1,), (0,))) + b3[...]))
        return h

    i3t = branch(hist_ref[0:1, :], w1i, b1i, w2i, b2i, w3i, b3i)
    e3t = branch(hist_ref[1:2, :], w1e, b1e, w2e, b2e, w3e, b3e)

    g_iota = lax.broadcasted_iota(jnp.int32, (_G, _N), 0)
    m = (batch_ref[...] == g_iota).astype(f32)
    cnt = jnp.maximum(mm(jnp.ones((1, _N), f32), m, ((1,), (1,))), 1.0)
    ipgt = mm(i3t, m, ((1,), (1,))) / cnt
    epgt = mm(e3t, m, ((1,), (1,))) / cnt
    ut = jnp.concatenate([ipgt, epgt], axis=0)
    o1t = relu(mm(fc1w[...], ut, ((1,), (0,))) + fc1b[...])
    out_ref[...] = mm(o1t, fc2w[...], ((0,), (1,))) + fc2b[...]


def kernel(x, internal_edge_index, internal_edge_attr, edge_index, edge_attr,
           batch, params):
    del internal_edge_attr, edge_attr

    ones_c = jnp.ones((_EPT,), jnp.float32)
    zeros_c = jnp.zeros((_N,), jnp.float32)
    hist = _hist_kernel()(internal_edge_index.astype(jnp.int32).reshape(-1),
                          edge_index.astype(jnp.int32).reshape(-1),
                          ones_c, zeros_c)

    p = params
    args = (
        x, hist, batch.astype(jnp.int32).reshape(1, _N),
        p["iml1"]["fn_w"], p["iml1"]["fn_b"].reshape(-1, 1),
        p["iml2"]["fn_w"], p["iml2"]["fn_b"].reshape(-1, 1),
        p["iml3"]["fn_w"], p["iml3"]["fn_b"].reshape(-1, 1),
        p["eml1"]["fn_w"], p["eml1"]["fn_b"].reshape(-1, 1),
        p["eml2"]["fn_w"], p["eml2"]["fn_b"].reshape(-1, 1),
        p["eml3"]["fn_w"], p["eml3"]["fn_b"].reshape(-1, 1),
        p["fc1_w"], p["fc1_b"].reshape(-1, 1),
        p["fc2_w"], p["fc2_b"].reshape(1, -1),
    )
    return pl.pallas_call(
        _dense_body,
        out_shape=jax.ShapeDtypeStruct((_G, 8), jnp.float32),
    )(*args)


# speedup vs baseline: 243.1274x; 1.0063x over previous
import functools

import jax
import jax.numpy as jnp
from jax import lax
from jax.experimental import pallas as pl
from jax.experimental.pallas import tpu as pltpu
from jax.experimental.pallas import tpu_sc as plsc

_N = 10000
_E = 320000
_G = 16

_NC = 2
_NS = 16
_L = 16

_EPT = _E // _NS


def _hist_body(iei_hbm, eei_hbm, ones_hbm, zeros_hbm, out_hbm,
               idx_v, ones_v, hist_sh):
    c = lax.axis_index("c")
    s = lax.axis_index("s")

    pltpu.sync_copy(ones_hbm, ones_v)

    @pl.when(c == 0)
    def _load_internal():
        pltpu.sync_copy(iei_hbm.at[pl.ds(s * _EPT, _EPT)], idx_v)

    @pl.when(c == 1)
    def _load_external():
        pltpu.sync_copy(eei_hbm.at[pl.ds(s * _EPT, _EPT)], idx_v)

    @pl.when(s == 0)
    def _zero_hist():
        pltpu.sync_copy(zeros_hbm, hist_sh)

    plsc.subcore_barrier()

    pltpu.sync_copy(ones_v, hist_sh.at[idx_v], add=True)

    plsc.subcore_barrier()

    @pl.when(s == 0)
    def _writeback():
        pltpu.sync_copy(hist_sh, out_hbm.at[c])


@functools.cache
def _hist_kernel():
    return pl.kernel(
        _hist_body,
        out_type=jax.ShapeDtypeStruct((_NC, _N), jnp.float32),
        mesh=plsc.VectorSubcoreMesh(
            core_axis_name="c", subcore_axis_name="s",
            num_cores=_NC, num_subcores=_NS),
        scratch_types=[
            pltpu.VMEM((_EPT,), jnp.int32),
            pltpu.VMEM((_EPT,), jnp.float32),
            pltpu.VMEM_SHARED((_N,), jnp.float32),
        ],
    )


def _dense_body(x_ref, hist_ref, batch_ref,
                w1i, b1i, w2i, b2i, w3i, b3i,
                w1e, b1e, w2e, b2e, w3e, b3e,
                fc1w, fc1b, fc2w, fc2b, out_ref):
    x = x_ref[...]
    relu = jax.nn.relu
    f32 = jnp.float32

    def mm(a, b, dims):
        return lax.dot_general(a, b, (dims, ((), ())),
                               preferred_element_type=f32)

    def col(b):
        return jnp.swapaxes(b[...], 0, 1)

    def branch(d_row, w1, b1, w2, b2, w3, b3):
        h = relu(d_row * (mm(w1[...], x, ((1,), (1,))) + col(b1)))
        h = relu(d_row * (mm(w2[...], h, ((1,), (0,))) + col(b2)))
        h = relu(d_row * (mm(w3[...], h, ((1,), (0,))) + col(b3)))
        return h

    i3t = branch(hist_ref[0:1, :], w1i, b1i, w2i, b2i, w3i, b3i)
    e3t = branch(hist_ref[1:2, :], w1e, b1e, w2e, b2e, w3e, b3e)

    g_iota = lax.broadcasted_iota(jnp.int32, (_G, _N), 0)
    m = (batch_ref[...] == g_iota).astype(f32)
    cnt = jnp.maximum(mm(jnp.ones((1, _N), f32), m, ((1,), (1,))), 1.0)
    ipgt = mm(i3t, m, ((1,), (1,))) / cnt
    epgt = mm(e3t, m, ((1,), (1,))) / cnt
    ut = jnp.concatenate([ipgt, epgt], axis=0)
    o1t = relu(mm(fc1w[...], ut, ((1,), (0,))) + col(fc1b))
    out_ref[...] = mm(o1t, fc2w[...], ((0,), (1,))) + fc2b[...]


def kernel(x, internal_edge_index, internal_edge_attr, edge_index, edge_attr,
           batch, params):
    del internal_edge_attr, edge_attr

    ones_c = jnp.ones((_EPT,), jnp.float32)
    zeros_c = jnp.zeros((_N,), jnp.float32)
    hist = _hist_kernel()(internal_edge_index.astype(jnp.int32).reshape(-1),
                          edge_index.astype(jnp.int32).reshape(-1),
                          ones_c, zeros_c)

    p = params
    args = (
        x, hist, batch.astype(jnp.int32).reshape(1, _N),
        p["iml1"]["fn_w"], p["iml1"]["fn_b"].reshape(1, -1),
        p["iml2"]["fn_w"], p["iml2"]["fn_b"].reshape(1, -1),
        p["iml3"]["fn_w"], p["iml3"]["fn_b"].reshape(1, -1),
        p["eml1"]["fn_w"], p["eml1"]["fn_b"].reshape(1, -1),
        p["eml2"]["fn_w"], p["eml2"]["fn_b"].reshape(1, -1),
        p["eml3"]["fn_w"], p["eml3"]["fn_b"].reshape(1, -1),
        p["fc1_w"], p["fc1_b"].reshape(1, -1),
        p["fc2_w"], p["fc2_b"].reshape(1, -1),
    )
    return pl.pallas_call(
        _dense_body,
        out_shape=jax.ShapeDtypeStruct((_G, 8), jnp.float32),
    )(*args)
